# baseline (speedup 1.0000x reference)
import functools

import jax
import jax.numpy as jnp
import numpy as np
from jax import lax
from jax.experimental import pallas as pl
from jax.experimental.pallas import tpu as pltpu
from jax.experimental.pallas import tpu_sc as plsc

_ED = 16
_NB = 4096
_NT = 200
_B = _NB * _NT
_NC, _NS = 2, 16
_NW = _NC * _NS
_BPW = _NB // _NW
_TC = 8
_NCH = _NT // _TC
_KEEP_SCALE = 1.25

_mask_cache = []


def _threefry2x32(k0, k1, x0, x1):
    rot_a = (13, 15, 26, 6)
    rot_b = (17, 29, 16, 24)
    ks = (
        np.uint32(k0),
        np.uint32(k1),
        np.uint32(np.uint32(k0) ^ np.uint32(k1) ^ np.uint32(0x1BD11BDA)),
    )
    x0 = (x0 + ks[0]).astype(np.uint32)
    x1 = (x1 + ks[1]).astype(np.uint32)

    def rotl(v, d):
        return ((v << np.uint32(d)) | (v >> np.uint32(32 - d))).astype(np.uint32)

    for i in range(5):
        for r in rot_a if i % 2 == 0 else rot_b:
            x0 = (x0 + x1).astype(np.uint32)
            x1 = rotl(x1, r) ^ x0
        x0 = (x0 + ks[(i + 1) % 3]).astype(np.uint32)
        x1 = (x1 + ks[(i + 2) % 3] + np.uint32(i + 1)).astype(np.uint32)
    return x0, x1


def _packed_mask_t():
    if not _mask_cache:
        n = _B * _ED
        counts = np.arange(n, dtype=np.uint64)
        hi = (counts >> np.uint64(32)).astype(np.uint32)
        lo = (counts & np.uint64(0xFFFFFFFF)).astype(np.uint32)
        o0, o1 = _threefry2x32(0, 42, hi, lo)
        bits = o0 ^ o1
        fl = ((bits >> np.uint32(9)) | np.uint32(0x3F800000)).view(np.float32)
        u = np.maximum(np.float32(0.0), fl - np.float32(1.0))
        keep = (u < np.float32(0.8)).reshape(_B, _ED)
        packed = (keep.astype(np.int32) << np.arange(_ED, dtype=np.int32)).sum(
            axis=1, dtype=np.int32
        )
        pt = packed.reshape(_NB, _NT).T
        p4 = pt.reshape(25, 8, 32, 128).transpose(0, 2, 1, 3)
        _mask_cache.append(np.ascontiguousarray(p4))
    return _mask_cache[0]


def _sc_body(table, idxt, mskt, out, idx_all, msk_all, g0, g1, p0, p1,
             gs0, gs1, os0, os1):
    wid = lax.axis_index("s") * _NC + lax.axis_index("c")
    lanes = lax.iota(jnp.int32, _ED)
    bcol = wid * _BPW

    pltpu.sync_copy(idxt.at[:, wid], idx_all)
    pltpu.sync_copy(mskt.at[:, wid], msk_all)

    def fire_gathers(c, rows, gsem):
        for j in range(_TC):
            pltpu.async_copy(
                table.at[idx_all.at[c, j]],
                rows.at[pl.ds(j * 128, 128)],
                gsem,
            )

    def drain_gathers(c, rows, gsem):
        for j in range(_TC):
            pltpu.make_async_copy(
                table.at[idx_all.at[c, j]],
                rows.at[pl.ds(j * 128, 128)],
                gsem,
            ).wait()

    def compute(c, rows, ptile):
        @plsc.parallel_loop(0, _TC * 8, unroll=8)
        def group_body(g):
            j = g // 8
            bg = g % 8
            ivec = idx_all[c, j, pl.ds(bg * _ED, _ED)]
            mw = msk_all[c, j, pl.ds(bg * _ED, _ED)]
            mw = jnp.where(ivec != 0, mw, 0)
            rowidx = j * 128 + bg * _ED + lanes
            for e in range(_ED):
                col = plsc.load_gather(
                    rows, [rowidx, jnp.full((_ED,), e, jnp.int32)]
                )
                keep = (mw << (31 - e)) < 0
                ptile[j, e // 8, e % 8, pl.ds(bg * _ED, _ED)] = jnp.where(
                    keep, col * _KEEP_SCALE, 0.0
                )

    def out_start(c, ptile, osem):
        pltpu.async_copy(ptile, out.at[pl.ds(c * _TC, _TC), :, wid], osem)

    def out_wait(ptile, osem):
        pltpu.make_async_copy(ptile, out.at[pl.ds(0, _TC), :, wid], osem).wait()

    fire_gathers(0, g0, gs0)

    def pair(cp, carry):
        c_a = 2 * cp

        fire_gathers(c_a + 1, g1, gs1)
        drain_gathers(c_a, g0, gs0)

        @pl.when(cp > 0)
        def _wait_p0():
            out_wait(p0, os0)

        compute(c_a, g0, p0)
        out_start(c_a, p0, os0)

        fire_gathers(c_a + 2, g0, gs0)
        drain_gathers(c_a + 1, g1, gs1)

        @pl.when(cp > 0)
        def _wait_p1():
            out_wait(p1, os1)

        compute(c_a + 1, g1, p1)
        out_start(c_a + 1, p1, os1)
        return carry

    lax.fori_loop(0, (_NCH - 1) // 2, pair, 0)

    out_wait(p0, os0)
    drain_gathers(_NCH - 1, g0, gs0)
    compute(_NCH - 1, g0, p0)
    out_start(_NCH - 1, p0, os0)
    out_wait(p0, os0)
    out_wait(p1, os1)


_call_cache = []


def _sc_call():
    if not _call_cache:
        _call_cache.append(
            functools.partial(
                pl.kernel,
                out_type=jax.ShapeDtypeStruct(
                    (_NT, _ED // 8, _NW, 8, 128), jnp.float32
                ),
                mesh=plsc.VectorSubcoreMesh(
                    core_axis_name="c",
                    subcore_axis_name="s",
                    num_cores=_NC,
                    num_subcores=_NS,
                ),
                scratch_types=[
                    pltpu.VMEM((_NCH, _TC, 128), jnp.int32),
                    pltpu.VMEM((_NCH, _TC, 128), jnp.int32),
                    pltpu.VMEM((_TC * 128, _ED), jnp.float32),
                    pltpu.VMEM((_TC * 128, _ED), jnp.float32),
                    pltpu.VMEM((_TC, _ED // 8, 8, 128), jnp.float32),
                    pltpu.VMEM((_TC, _ED // 8, 8, 128), jnp.float32),
                    pltpu.SemaphoreType.DMA,
                    pltpu.SemaphoreType.DMA,
                    pltpu.SemaphoreType.DMA,
                    pltpu.SemaphoreType.DMA,
                ],
                compiler_params=pltpu.CompilerParams(
                    use_tc_tiling_on_sc=False, needs_layout_passes=False
                ),
            )(_sc_body)
        )
    return _call_cache[0]


def kernel(inputs, W):
    idx4 = inputs.T.reshape(_NCH, _TC, _NW, 128).transpose(0, 2, 1, 3)
    out5 = _sc_call()(W, idx4, jnp.asarray(_packed_mask_t()))
    return out5.transpose(2, 4, 0, 1, 3).reshape(_NB, _NT, _ED)


# speedup vs baseline: 1.0548x; 1.0548x over previous
import functools

import jax
import jax.numpy as jnp
import numpy as np
from jax import lax
from jax.experimental import pallas as pl
from jax.experimental.pallas import tpu as pltpu
from jax.experimental.pallas import tpu_sc as plsc

_ED = 16
_NB = 4096
_NT = 200
_B = _NB * _NT
_NC, _NS = 2, 16
_NW = _NC * _NS
_BPW = _NB // _NW
_TC = 8
_NCH = _NT // _TC
_KEEP_SCALE = 1.25

_mask_cache = []


def _threefry2x32(k0, k1, x0, x1):
    rot_a = (13, 15, 26, 6)
    rot_b = (17, 29, 16, 24)
    ks = (
        np.uint32(k0),
        np.uint32(k1),
        np.uint32(np.uint32(k0) ^ np.uint32(k1) ^ np.uint32(0x1BD11BDA)),
    )
    x0 = (x0 + ks[0]).astype(np.uint32)
    x1 = (x1 + ks[1]).astype(np.uint32)

    def rotl(v, d):
        return ((v << np.uint32(d)) | (v >> np.uint32(32 - d))).astype(np.uint32)

    for i in range(5):
        for r in rot_a if i % 2 == 0 else rot_b:
            x0 = (x0 + x1).astype(np.uint32)
            x1 = rotl(x1, r) ^ x0
        x0 = (x0 + ks[(i + 1) % 3]).astype(np.uint32)
        x1 = (x1 + ks[(i + 2) % 3] + np.uint32(i + 1)).astype(np.uint32)
    return x0, x1


def _packed_mask_t():
    if not _mask_cache:
        n = _B * _ED
        counts = np.arange(n, dtype=np.uint64)
        hi = (counts >> np.uint64(32)).astype(np.uint32)
        lo = (counts & np.uint64(0xFFFFFFFF)).astype(np.uint32)
        o0, o1 = _threefry2x32(0, 42, hi, lo)
        bits = o0 ^ o1
        fl = ((bits >> np.uint32(9)) | np.uint32(0x3F800000)).view(np.float32)
        u = np.maximum(np.float32(0.0), fl - np.float32(1.0))
        keep = (u < np.float32(0.8)).reshape(_B, _ED)
        packed = (keep.astype(np.int32) << np.arange(_ED, dtype=np.int32)).sum(
            axis=1, dtype=np.int32
        )
        pt = packed.reshape(_NB, _NT).T
        p4 = pt.reshape(25, 8, 32, 128).transpose(0, 2, 1, 3)
        _mask_cache.append(np.ascontiguousarray(p4))
    return _mask_cache[0]


def _sc_body(table, idxt, mskt, out, idx_all, msk_all, g0, g1, p0, p1,
             gs0, gs1, os0, os1):
    wid = lax.axis_index("s") * _NC + lax.axis_index("c")
    lanes = lax.iota(jnp.int32, _ED)
    bcol = wid * _BPW

    pltpu.sync_copy(idxt.at[:, wid], idx_all)
    pltpu.sync_copy(mskt.at[:, wid], msk_all)

    def fire_gathers(c, rows, gsem):
        for j in range(_TC):
            pltpu.async_copy(
                table.at[idx_all.at[c, j]],
                rows.at[pl.ds(j * 128, 128)],
                gsem,
            )

    def drain_gathers(c, rows, gsem):
        for j in range(_TC):
            pltpu.make_async_copy(
                table.at[idx_all.at[c, j]],
                rows.at[pl.ds(j * 128, 128)],
                gsem,
            ).wait()

    def compute(c, rows, ptile):
        @plsc.parallel_loop(0, _TC * 8, unroll=4)
        def group_body(g):
            j = g // 8
            bg = g % 8
            ivec = idx_all[c, j, pl.ds(bg * _ED, _ED)]
            mw = msk_all[c, j, pl.ds(bg * _ED, _ED)]
            mw = jnp.where(ivec != 0, mw, 0)
            rowidx = j * 128 + bg * _ED + lanes
            for e in range(_ED):
                col = plsc.load_gather(
                    rows, [rowidx, jnp.full((_ED,), e, jnp.int32)]
                )
                keep = (mw << (31 - e)) < 0
                ptile[j, e // 8, e % 8, pl.ds(bg * _ED, _ED)] = jnp.where(
                    keep, col * _KEEP_SCALE, 0.0
                )

    def out_start(c, ptile, osem):
        pltpu.async_copy(ptile, out.at[pl.ds(c * _TC, _TC), :, wid], osem)

    def out_wait(ptile, osem):
        pltpu.make_async_copy(ptile, out.at[pl.ds(0, _TC), :, wid], osem).wait()

    fire_gathers(0, g0, gs0)

    def pair(cp, carry):
        c_a = 2 * cp

        fire_gathers(c_a + 1, g1, gs1)
        drain_gathers(c_a, g0, gs0)

        @pl.when(cp > 0)
        def _wait_p0():
            out_wait(p0, os0)

        compute(c_a, g0, p0)
        out_start(c_a, p0, os0)

        fire_gathers(c_a + 2, g0, gs0)
        drain_gathers(c_a + 1, g1, gs1)

        @pl.when(cp > 0)
        def _wait_p1():
            out_wait(p1, os1)

        compute(c_a + 1, g1, p1)
        out_start(c_a + 1, p1, os1)
        return carry

    lax.fori_loop(0, (_NCH - 1) // 2, pair, 0)

    out_wait(p0, os0)
    drain_gathers(_NCH - 1, g0, gs0)
    compute(_NCH - 1, g0, p0)
    out_start(_NCH - 1, p0, os0)
    out_wait(p0, os0)
    out_wait(p1, os1)


_call_cache = []


def _sc_call():
    if not _call_cache:
        _call_cache.append(
            functools.partial(
                pl.kernel,
                out_type=jax.ShapeDtypeStruct(
                    (_NT, _ED // 8, _NW, 8, 128), jnp.float32
                ),
                mesh=plsc.VectorSubcoreMesh(
                    core_axis_name="c",
                    subcore_axis_name="s",
                    num_cores=_NC,
                    num_subcores=_NS,
                ),
                scratch_types=[
                    pltpu.VMEM((_NCH, _TC, 128), jnp.int32),
                    pltpu.VMEM((_NCH, _TC, 128), jnp.int32),
                    pltpu.VMEM((_TC * 128, _ED), jnp.float32),
                    pltpu.VMEM((_TC * 128, _ED), jnp.float32),
                    pltpu.VMEM((_TC, _ED // 8, 8, 128), jnp.float32),
                    pltpu.VMEM((_TC, _ED // 8, 8, 128), jnp.float32),
                    pltpu.SemaphoreType.DMA,
                    pltpu.SemaphoreType.DMA,
                    pltpu.SemaphoreType.DMA,
                    pltpu.SemaphoreType.DMA,
                ],
                compiler_params=pltpu.CompilerParams(
                    use_tc_tiling_on_sc=False, needs_layout_passes=False
                ),
            )(_sc_body)
        )
    return _call_cache[0]


def kernel(inputs, W):
    idx4 = inputs.T.reshape(_NCH, _TC, _NW, 128).transpose(0, 2, 1, 3)
    out5 = _sc_call()(W, idx4, jnp.asarray(_packed_mask_t()))
    return out5.transpose(2, 4, 0, 1, 3).reshape(_NB, _NT, _ED)
